# baseline (speedup 1.0000x reference)
import jax
import jax.numpy as jnp
from jax import lax
from jax.experimental import pallas as pl
from jax.experimental.pallas import tpu as pltpu
from jax.experimental.pallas import tpu_sc as plsc

B = 128
N = 1_000_000
NC, NS, L = 2, 16, 16
NW = NC * NS
RPW = B // NW
D = 2048
NIDX = 16
TROWS = B * N // D
WPW = 122


def _probe_body(pred, tgt, out, buf0, buf1, stage, sem0, sem1):
    cid = lax.axis_index("c")
    sid = lax.axis_index("s")
    wid = cid * NS + sid
    base_row = wid * 1953
    lane = lax.iota(jnp.int32, L)

    bufs = (buf0, buf1)
    sems = (sem0, sem1)

    def gather_start(w, slot):
        idxv = lane + (base_row + w * NIDX)
        pltpu.async_copy(pred.at[idxv], bufs[slot], sems[slot])

    def gather_wait(w, slot):
        idxv = lane + (base_row + w * NIDX)
        pltpu.make_async_copy(pred.at[idxv], bufs[slot], sems[slot]).wait()

    gather_start(0, 0)

    def dstep(d_, acc):
        wA = 2 * d_
        gather_wait(wA, 0)
        gather_start(wA + 1, 1)
        acc = jnp.maximum(acc, buf0[0, pl.ds(0, L)])
        gather_wait(wA + 1, 1)
        gather_start(jnp.minimum(wA + 2, WPW - 1), 0)
        acc = jnp.maximum(acc, buf1[0, pl.ds(0, L)])
        return acc

    acc = lax.fori_loop(0, WPW // 2, dstep, jnp.zeros((L,), jnp.float32))
    gather_wait(WPW - 1, 0)

    stage[...] = acc
    pltpu.sync_copy(stage, out.at[wid])


@jax.jit
def kernel(pred, target):
    mesh = plsc.VectorSubcoreMesh(core_axis_name="c", subcore_axis_name="s")
    run = pl.kernel(
        _probe_body,
        out_type=jax.ShapeDtypeStruct((NW, L), jnp.float32),
        mesh=mesh,
        scratch_types=[
            pltpu.VMEM((NIDX, D), jnp.float32),
            pltpu.VMEM((NIDX, D), jnp.float32),
            pltpu.VMEM((L,), jnp.float32),
            pltpu.SemaphoreType.DMA,
            pltpu.SemaphoreType.DMA,
        ],
    )
    partials = run(pred.reshape(TROWS, D), target.astype(jnp.int32))
    return jnp.sum(partials[:, :1], axis=0)


# speedup vs baseline: 16.9238x; 16.9238x over previous
import jax
import jax.numpy as jnp
from jax.experimental import pallas as pl
from jax.experimental.pallas import tpu as pltpu

B = 128
N = 1_000_000
BW = 8_192
GRID = -(-N // BW)
BIG_I32 = 2**31 - 1


def _acc_body(tgt_ref, x_ref, out_ref, m_s, i_s):
    j = pl.program_id(0)
    cols = jax.lax.broadcasted_iota(jnp.int32, (B, BW), 1) + j * BW
    x = jnp.where(cols < N, x_ref[...], -jnp.inf)
    bm = jnp.max(x, axis=1)
    masked = jnp.where(x == bm[:, None], cols, BIG_I32)
    bi = jnp.min(masked, axis=1)

    @pl.when(j == 0)
    def _():
        m_s[...] = bm
        i_s[...] = bi

    @pl.when(j > 0)
    def _():
        better = bm > m_s[...]
        m_s[...] = jnp.where(better, bm, m_s[...])
        i_s[...] = jnp.where(better, bi, i_s[...])

    @pl.when(j == GRID - 1)
    def _():
        t = tgt_ref[0, :]
        out_ref[...] = jnp.sum((i_s[...] == t).astype(jnp.float32)).reshape(1, 1)


@jax.jit
def kernel(pred, target):
    out = pl.pallas_call(
        _acc_body,
        grid=(GRID,),
        in_specs=[
            pl.BlockSpec((1, B), lambda j: (0, 0)),
            pl.BlockSpec((B, BW), lambda j: (0, j)),
        ],
        out_specs=pl.BlockSpec((1, 1), lambda j: (0, 0)),
        out_shape=jax.ShapeDtypeStruct((1, 1), jnp.float32),
        scratch_shapes=[
            pltpu.VMEM((B,), jnp.float32),
            pltpu.VMEM((B,), jnp.int32),
        ],
    )(target.astype(jnp.int32).reshape(1, B), pred)
    return out.reshape(1)


# speedup vs baseline: 17.4384x; 1.0304x over previous
import jax
import jax.numpy as jnp
from jax.experimental import pallas as pl
from jax.experimental.pallas import tpu as pltpu

B = 128
N = 1_000_000
BW = 8_192
GRID = -(-N // BW)
BIG_I32 = 2**31 - 1


def _acc_body(tgt_ref, x_ref, out_ref, m_s, i_s):
    j = pl.program_id(0)

    def scan_block(x):
        cols = jax.lax.broadcasted_iota(jnp.int32, (B, BW), 1) + j * BW
        bm = jnp.max(x, axis=1)
        masked = jnp.where(x == bm[:, None], cols, BIG_I32)
        bi = jnp.min(masked, axis=1)
        better = (bm > m_s[...]) | (j == 0)
        m_s[...] = jnp.where(better, bm, m_s[...])
        i_s[...] = jnp.where(better, bi, i_s[...])

    @pl.when(j < GRID - 1)
    def _():
        scan_block(x_ref[...])

    @pl.when(j == GRID - 1)
    def _():
        cols = jax.lax.broadcasted_iota(jnp.int32, (B, BW), 1) + j * BW
        scan_block(jnp.where(cols < N, x_ref[...], -jnp.inf))

    @pl.when(j == GRID - 1)
    def _():
        t = tgt_ref[0, :]
        out_ref[...] = jnp.sum((i_s[...] == t).astype(jnp.float32)).reshape(1, 1)


@jax.jit
def kernel(pred, target):
    out = pl.pallas_call(
        _acc_body,
        grid=(GRID,),
        in_specs=[
            pl.BlockSpec((1, B), lambda j: (0, 0)),
            pl.BlockSpec((B, BW), lambda j: (0, j)),
        ],
        out_specs=pl.BlockSpec((1, 1), lambda j: (0, 0)),
        out_shape=jax.ShapeDtypeStruct((1, 1), jnp.float32),
        scratch_shapes=[
            pltpu.VMEM((B,), jnp.float32),
            pltpu.VMEM((B,), jnp.int32),
        ],
    )(target.astype(jnp.int32).reshape(1, B), pred)
    return out.reshape(1)
